# baseline (device time: 11160 ns/iter reference)
import jax
import jax.numpy as jnp
from jax import lax
from jax.experimental import pallas as pl
from jax.experimental.pallas import tpu as pltpu

N_DEV = 4
S = 2

RAW_A = 0
RAW_B = 1
DIR_A = 2
DIR_B = 3
PART_A = 4
PART_B = 5

S_RAW_A = 0
S_RAW_B = S
S_DIR_A = 2 * S
S_DIR_B = 2 * S + 1
S_PART_A = 2 * S + 2
S_PART_B = 3 * S + 2
N_SEM = 4 * S + 2


def kernel(x):
    _, m, n_total = x.shape
    n_per = n_total // N_DEV
    h = m // 2
    hp = h // S

    def body(x_ref, out_ref, recv_buf, stage_buf, send_sems, recv_sems):
        my = lax.axis_index("i")
        left = lax.rem(my + N_DEV - 1, N_DEV)
        right = lax.rem(my + 1, N_DEV)
        c_left = left * n_per
        c_right = right * n_per
        c_opp = lax.rem(my + 2, N_DEV) * n_per
        c_mine = my * n_per

        barrier_sem = pltpu.get_barrier_semaphore()
        for nbr in [left, right]:
            pl.semaphore_signal(
                barrier_sem, inc=1,
                device_id=(nbr,), device_id_type=pl.DeviceIdType.MESH,
            )
        pl.semaphore_wait(barrier_sem, 2)

        def copy(src, slot, rows, sem_idx, dst_dev):
            return pltpu.make_async_remote_copy(
                src_ref=src,
                dst_ref=recv_buf.at[slot, rows, :],
                send_sem=send_sems.at[sem_idx],
                recv_sem=recv_sems.at[sem_idx],
                device_id=(dst_dev,),
                device_id_type=pl.DeviceIdType.MESH,
            )

        raws = []
        for s in range(S):
            ra = copy(x_ref.at[0, pl.ds(s * hp, hp), pl.ds(c_opp, n_per)],
                      RAW_A, pl.ds(s * hp, hp), S_RAW_A + s, left)
            rb = copy(x_ref.at[0, pl.ds(h + s * hp, hp), pl.ds(c_opp, n_per)],
                      RAW_B, pl.ds(s * hp, hp), S_RAW_B + s, right)
            ra.start()
            rb.start()
            raws.append((ra, rb))

        dir_a = copy(x_ref.at[0, pl.ds(0, h), pl.ds(c_right, n_per)],
                     DIR_A, pl.ds(0, h), S_DIR_A, right)
        dir_b = copy(x_ref.at[0, pl.ds(h, h), pl.ds(c_left, n_per)],
                     DIR_B, pl.ds(0, h), S_DIR_B, left)
        dir_a.start()
        dir_b.start()

        parts = []
        for s in range(S):
            ra, rb = raws[s]
            rows = pl.ds(s * hp, hp)
            ra.wait_recv()
            stage_buf[0, rows, :] = (
                x_ref[0, pl.ds(s * hp, hp), pl.ds(c_left, n_per)]
                + recv_buf[RAW_A, rows, :]
            )
            pa = copy(stage_buf.at[0, rows, :], PART_A, rows,
                      S_PART_A + s, left)
            pa.start()
            rb.wait_recv()
            stage_buf[1, rows, :] = (
                x_ref[0, pl.ds(h + s * hp, hp), pl.ds(c_right, n_per)]
                + recv_buf[RAW_B, rows, :]
            )
            pb = copy(stage_buf.at[1, rows, :], PART_B, rows,
                      S_PART_B + s, right)
            pb.start()
            parts.append((pa, pb))

        dir_a.wait_recv()
        acc_t = x_ref[0, pl.ds(0, h), pl.ds(c_mine, n_per)] + recv_buf[DIR_A]
        dir_b.wait_recv()
        acc_b = x_ref[0, pl.ds(h, h), pl.ds(c_mine, n_per)] + recv_buf[DIR_B]
        for s in range(S):
            pa, pb = parts[s]
            rows = pl.ds(s * hp, hp)
            pa.wait_recv()
            out_ref[rows, :] = (
                acc_t[s * hp:(s + 1) * hp, :] + recv_buf[PART_A, rows, :]
            )
            pb.wait_recv()
            out_ref[pl.ds(h + s * hp, hp), :] = (
                acc_b[s * hp:(s + 1) * hp, :] + recv_buf[PART_B, rows, :]
            )

        for ra, rb in raws:
            ra.wait_send()
            rb.wait_send()
        dir_a.wait_send()
        dir_b.wait_send()
        for pa, pb in parts:
            pa.wait_send()
            pb.wait_send()

    return pl.pallas_call(
        body,
        out_shape=jax.ShapeDtypeStruct((m, n_per), x.dtype),
        in_specs=[pl.BlockSpec(memory_space=pltpu.VMEM)],
        out_specs=pl.BlockSpec(memory_space=pltpu.VMEM),
        scratch_shapes=[
            pltpu.VMEM((6, h, n_per), x.dtype),
            pltpu.VMEM((2, h, n_per), x.dtype),
            pltpu.SemaphoreType.DMA((N_SEM,)),
            pltpu.SemaphoreType.DMA((N_SEM,)),
        ],
        compiler_params=pltpu.CompilerParams(collective_id=0),
    )(x)


# device time: 11102 ns/iter; 1.0052x vs baseline; 1.0052x over previous
import jax
from jax import lax
from jax.experimental import pallas as pl
from jax.experimental.pallas import tpu as pltpu

N_DEV = 4
S = 1

RAW_A = 0
RAW_B = 1
DIR_A = 2
DIR_B = 3
PART_A = 4
PART_B = 5

S_RAW_A = 0
S_RAW_B = S
S_DIR_A = 2 * S
S_DIR_B = 2 * S + 1
S_PART_A = 2 * S + 2
S_PART_B = 3 * S + 2
N_SEM = 4 * S + 2


def kernel(x):
    _, m, n_total = x.shape
    n_per = n_total // N_DEV
    h = m // 2
    hp = h // S

    def body(x_ref, out_ref, recv_buf, stage_buf, send_sems, recv_sems):
        my = lax.axis_index("i")
        left = lax.rem(my + N_DEV - 1, N_DEV)
        right = lax.rem(my + 1, N_DEV)
        c_left = left * n_per
        c_right = right * n_per
        c_opp = lax.rem(my + 2, N_DEV) * n_per
        c_mine = my * n_per

        barrier_sem = pltpu.get_barrier_semaphore()
        for nbr in [left, right]:
            pl.semaphore_signal(
                barrier_sem, inc=1,
                device_id=(nbr,), device_id_type=pl.DeviceIdType.MESH,
            )
        pl.semaphore_wait(barrier_sem, 2)

        def copy(src, slot, rows, sem_idx, dst_dev):
            return pltpu.make_async_remote_copy(
                src_ref=src,
                dst_ref=recv_buf.at[slot, rows, :],
                send_sem=send_sems.at[sem_idx],
                recv_sem=recv_sems.at[sem_idx],
                device_id=(dst_dev,),
                device_id_type=pl.DeviceIdType.MESH,
            )

        raws = []
        for s in range(S):
            ra = copy(x_ref.at[0, pl.ds(s * hp, hp), pl.ds(c_opp, n_per)],
                      RAW_A, pl.ds(s * hp, hp), S_RAW_A + s, left)
            rb = copy(x_ref.at[0, pl.ds(h + s * hp, hp), pl.ds(c_opp, n_per)],
                      RAW_B, pl.ds(s * hp, hp), S_RAW_B + s, right)
            ra.start()
            rb.start()
            raws.append((ra, rb))

        dir_a = copy(x_ref.at[0, pl.ds(0, h), pl.ds(c_right, n_per)],
                     DIR_A, pl.ds(0, h), S_DIR_A, right)
        dir_b = copy(x_ref.at[0, pl.ds(h, h), pl.ds(c_left, n_per)],
                     DIR_B, pl.ds(0, h), S_DIR_B, left)
        dir_a.start()
        dir_b.start()

        parts = []
        for s in range(S):
            ra, rb = raws[s]
            rows = pl.ds(s * hp, hp)
            ra.wait_recv()
            stage_buf[0, rows, :] = (
                x_ref[0, pl.ds(s * hp, hp), pl.ds(c_left, n_per)]
                + recv_buf[RAW_A, rows, :]
            )
            pa = copy(stage_buf.at[0, rows, :], PART_A, rows,
                      S_PART_A + s, left)
            pa.start()
            rb.wait_recv()
            stage_buf[1, rows, :] = (
                x_ref[0, pl.ds(h + s * hp, hp), pl.ds(c_right, n_per)]
                + recv_buf[RAW_B, rows, :]
            )
            pb = copy(stage_buf.at[1, rows, :], PART_B, rows,
                      S_PART_B + s, right)
            pb.start()
            parts.append((pa, pb))

        dir_a.wait_recv()
        acc_t = x_ref[0, pl.ds(0, h), pl.ds(c_mine, n_per)] + recv_buf[DIR_A]
        dir_b.wait_recv()
        acc_b = x_ref[0, pl.ds(h, h), pl.ds(c_mine, n_per)] + recv_buf[DIR_B]
        for s in range(S):
            pa, pb = parts[s]
            rows = pl.ds(s * hp, hp)
            pa.wait_recv()
            out_ref[rows, :] = (
                acc_t[s * hp:(s + 1) * hp, :] + recv_buf[PART_A, rows, :]
            )
            pb.wait_recv()
            out_ref[pl.ds(h + s * hp, hp), :] = (
                acc_b[s * hp:(s + 1) * hp, :] + recv_buf[PART_B, rows, :]
            )

        for ra, rb in raws:
            ra.wait_send()
            rb.wait_send()
        dir_a.wait_send()
        dir_b.wait_send()
        for pa, pb in parts:
            pa.wait_send()
            pb.wait_send()

    return pl.pallas_call(
        body,
        out_shape=jax.ShapeDtypeStruct((m, n_per), x.dtype),
        in_specs=[pl.BlockSpec(memory_space=pltpu.VMEM)],
        out_specs=pl.BlockSpec(memory_space=pltpu.VMEM),
        scratch_shapes=[
            pltpu.VMEM((6, h, n_per), x.dtype),
            pltpu.VMEM((2, h, n_per), x.dtype),
            pltpu.SemaphoreType.DMA((N_SEM,)),
            pltpu.SemaphoreType.DMA((N_SEM,)),
        ],
        compiler_params=pltpu.CompilerParams(collective_id=0),
    )(x)
